# baseline (speedup 1.0000x reference)
import functools

import jax
import jax.numpy as jnp
from jax import lax
from jax.experimental import pallas as pl
from jax.experimental.pallas import tpu as pltpu
from jax.experimental.pallas import tpu_sc as plsc

N = 10000
F = 128
HID = 256
E = 320000

NC = 2
NS = 16
NW = NC * NS

NPAD = 10112
STRIPE = NPAD // NS
C = 128
EPAD = 327680
KD = EPAD // NW // C
U = 4
CU = C // U
RF = 120
RS = 2 * KD - RF
PH_F = (64, 56)
PH_S = (24, 16)
QMAX = max(PH_F)
NROWS = EPAD // C
DPAD = 10240
DSTRIPE = DPAD // NS

ROWB = 632
GRID = NPAD // ROWB


@functools.cache
def _sc_kernels():
    mesh = plsc.VectorSubcoreMesh(core_axis_name="c", subcore_axis_name="s")

    @functools.partial(
        pl.kernel,
        out_type=jax.ShapeDtypeStruct((NC, NS, DSTRIPE), jnp.float32),
        mesh=mesh,
        scratch_types=[
            pltpu.VMEM((KD, 128), jnp.int32),
            pltpu.VMEM((128,), jnp.float32),
            pltpu.VMEM_SHARED((DPAD,), jnp.float32),
        ],
    )
    def sc_degree(dst_hbm, zeros_hbm, ones_hbm, out_hbm, dst_v, ones_v, acc):
        c = lax.axis_index("c")
        s = lax.axis_index("s")
        wid = s * NC + c
        pltpu.sync_copy(dst_hbm.at[wid], dst_v)
        pltpu.sync_copy(ones_hbm, ones_v)
        pltpu.sync_copy(zeros_hbm, acc.at[pl.ds(s * DSTRIPE, DSTRIPE)])
        plsc.subcore_barrier()

        @pl.loop(0, KD)
        def _(j):
            pltpu.sync_copy(ones_v, acc.at[dst_v.at[j]], add=True)

        plsc.subcore_barrier()
        pltpu.sync_copy(acc.at[pl.ds(s * DSTRIPE, DSTRIPE)], out_hbm.at[c, s])

    @functools.partial(
        pl.kernel,
        out_type=jax.ShapeDtypeStruct((NC, NS, STRIPE, F), jnp.float32),
        mesh=mesh,
        scratch_types=[
            pltpu.VMEM((QMAX, C), jnp.int32),
            pltpu.VMEM((QMAX, C), jnp.int32),
            pltpu.VMEM((C, F), jnp.float32),
            pltpu.VMEM((C, F), jnp.float32),
            pltpu.VMEM_SHARED((NPAD, F), jnp.float32),
            pltpu.SemaphoreType.DMA,
            pltpu.SemaphoreType.DMA,
            pltpu.SemaphoreType.DMA,
            pltpu.SemaphoreType.DMA,
        ],
    )
    def sc_aggregate(g_hbm, src_hbm, dst_hbm, zeros_hbm, out_hbm,
                     src_v, dst_v, buf0, buf1, acc, sg0, sg1, ss0, ss1):
        c = lax.axis_index("c")
        s = lax.axis_index("s")
        pltpu.sync_copy(zeros_hbm, acc.at[pl.ds(s * STRIPE, STRIPE)])
        plsc.subcore_barrier()

        bufs = (buf0, buf1)
        sem_g = (sg0, sg1)
        sem_s = (ss0, ss1)

        def gathers(q, b):
            for u in range(U):
                pltpu.async_copy(
                    g_hbm.at[src_v.at[q, pl.ds(u * CU, CU)]],
                    bufs[b].at[pl.ds(u * CU, CU)], sem_g[b])

        def wait_gathers(b):
            for u in range(U):
                pltpu.make_async_copy(
                    g_hbm.at[src_v.at[0, pl.ds(0, CU)]],
                    bufs[b].at[pl.ds(0, CU)], sem_g[b]).wait()

        def scatter(q, b):
            pltpu.async_copy(bufs[b], acc.at[dst_v.at[q]], sem_s[b], add=True)

        def wait_scatter(b):
            pltpu.make_async_copy(bufs[b], acc.at[dst_v.at[0]],
                                  sem_s[b]).wait()

        def pipeline(base, phases):
            off = 0
            for Q in phases:
                pltpu.sync_copy(src_hbm.at[pl.ds(base + off, Q)],
                                src_v.at[pl.ds(0, Q)])
                pltpu.sync_copy(dst_hbm.at[pl.ds(base + off, Q)],
                                dst_v.at[pl.ds(0, Q)])
                off += Q
                gathers(0, 0)
                wait_gathers(0)
                scatter(0, 0)
                gathers(1, 1)
                @pl.loop(0, (Q - 2) // 2)
                def _(i):
                    for b in (1, 0):
                        q = 2 * i + 2 - b
                        wait_gathers(b)
                        scatter(q, b)
                        wait_scatter(1 - b)
                        gathers(q + 1, 1 - b)
                wait_gathers(1)
                scatter(Q - 1, 1)
                wait_scatter(0)
                wait_scatter(1)

        @pl.when(c == 0)
        def _():
            pipeline(s * RF, PH_F)

        @pl.when(c == 1)
        def _():
            pipeline(NS * RF + s * RS, PH_S)

        plsc.subcore_barrier()
        pltpu.sync_copy(acc.at[pl.ds(s * STRIPE, STRIPE)], out_hbm.at[c, s])

    return sc_degree, sc_aggregate


def _sc_degree(*args):
    return _sc_kernels()[0](*args)


def _sc_aggregate(*args):
    return _sc_kernels()[1](*args)


def _tc_prescale_kernel(d0_ref, d1_ref, x_ref, g_ref, dinv_ref, dinv2_ref):
    deg = d0_ref[...] + d1_ref[...] + 1.0
    dinv = lax.rsqrt(deg)
    dinv2 = 1.0 / deg
    dinv_ref[...] = dinv
    dinv2_ref[...] = dinv2
    g_ref[...] = x_ref[...] * dinv


def _tc_mid_kernel(a0_ref, a1_ref, x_ref, dinv_ref, dinv2_ref,
                   w1_ref, b1_ref, w2_ref, t2_ref, g2_ref):
    dinv = dinv_ref[...]
    p = dinv * (a0_ref[...] + a1_ref[...]) + dinv2_ref[...] * x_ref[...]
    h = jnp.maximum(
        jnp.dot(p, w1_ref[...], preferred_element_type=jnp.float32)
        + b1_ref[...], 0.0)
    t2 = jnp.dot(h, w2_ref[...], preferred_element_type=jnp.float32)
    t2_ref[...] = t2
    g2_ref[...] = t2 * dinv


def _tc_final_kernel(a0_ref, a1_ref, t2_ref, dinv_ref, dinv2_ref, b2_ref,
                     out_ref):
    o = (dinv_ref[...] * (a0_ref[...] + a1_ref[...])
         + dinv2_ref[...] * t2_ref[...] + b2_ref[...])
    m = jnp.max(o, axis=1, keepdims=True)
    e = jnp.exp(o - m)
    lse = jnp.log(jnp.sum(e, axis=1, keepdims=True)) + m
    out_ref[...] = o - lse


def _row_spec(width):
    return pl.BlockSpec((ROWB, width), lambda i: (i, 0))


def _full_spec(shape):
    return pl.BlockSpec(shape, lambda i: tuple(0 for _ in shape))


def kernel(x, edge_index, W1, b1, W2, b2):
    src = edge_index[0].astype(jnp.int32)
    dst = edge_index[1].astype(jnp.int32)
    pad = EPAD - E
    srcp = jnp.concatenate([src, jnp.zeros((pad,), jnp.int32)])
    dstp = jnp.concatenate([dst, jnp.full((pad,), N, jnp.int32)])
    src_r = srcp.reshape(NROWS, C)
    dst_r = dstp.reshape(NROWS, C)
    dst3 = dstp.reshape(NW, KD, C)

    xp = jnp.zeros((NPAD, F), jnp.float32).at[:N].set(x)
    zeros1 = jnp.zeros((DSTRIPE,), jnp.float32)
    zeros2 = jnp.zeros((STRIPE, F), jnp.float32)
    ones_c = jnp.ones((128,), jnp.float32)

    degp = _sc_degree(dst3, zeros1, ones_c).reshape(NC, DPAD)[:, :NPAD]
    d0 = degp[0].reshape(NPAD, 1)
    d1 = degp[1].reshape(NPAD, 1)

    g1, dinv, dinv2 = pl.pallas_call(
        _tc_prescale_kernel,
        grid=(GRID,),
        in_specs=[_row_spec(1), _row_spec(1), _row_spec(F)],
        out_specs=[_row_spec(F), _row_spec(1), _row_spec(1)],
        out_shape=[
            jax.ShapeDtypeStruct((NPAD, F), jnp.float32),
            jax.ShapeDtypeStruct((NPAD, 1), jnp.float32),
            jax.ShapeDtypeStruct((NPAD, 1), jnp.float32),
        ],
    )(d0, d1, xp)

    agg1 = _sc_aggregate(g1, src_r, dst_r, zeros2).reshape(NC, NPAD, F)

    t2, g2 = pl.pallas_call(
        _tc_mid_kernel,
        grid=(GRID,),
        in_specs=[
            _row_spec(F), _row_spec(F), _row_spec(F),
            _row_spec(1), _row_spec(1),
            _full_spec((F, HID)), _full_spec((1, HID)), _full_spec((HID, F)),
        ],
        out_specs=[_row_spec(F), _row_spec(F)],
        out_shape=[
            jax.ShapeDtypeStruct((NPAD, F), jnp.float32),
            jax.ShapeDtypeStruct((NPAD, F), jnp.float32),
        ],
    )(agg1[0], agg1[1], xp, dinv, dinv2, W1, b1.reshape(1, HID), W2)

    agg2 = _sc_aggregate(g2, src_r, dst_r, zeros2).reshape(NC, NPAD, F)

    out = pl.pallas_call(
        _tc_final_kernel,
        grid=(GRID,),
        in_specs=[
            _row_spec(F), _row_spec(F), _row_spec(F),
            _row_spec(1), _row_spec(1), _full_spec((1, F)),
        ],
        out_specs=_row_spec(F),
        out_shape=jax.ShapeDtypeStruct((NPAD, F), jnp.float32),
    )(agg2[0], agg2[1], t2, dinv, dinv2, b2.reshape(1, F))

    return out[:N]


# speedup vs baseline: 1.0550x; 1.0550x over previous
import functools

import jax
import jax.numpy as jnp
from jax import lax
from jax.experimental import pallas as pl
from jax.experimental.pallas import tpu as pltpu
from jax.experimental.pallas import tpu_sc as plsc

N = 10000
F = 128
HID = 256
E = 320000

NC = 2
NS = 16
NW = NC * NS

NPAD = 10112
STRIPE = NPAD // NS
C = 128
EPAD = 327680
KD = EPAD // NW // C
U = 4
CU = C // U
RF = 120
RS = 2 * KD - RF
PH_F = (64, 56)
PH_S = (24, 16)
QMAX = max(PH_F)
NROWS = EPAD // C
DPAD = 10240
DSTRIPE = DPAD // NS

ROWB = 632
GRID = NPAD // ROWB


@functools.cache
def _sc_kernels():
    mesh = plsc.VectorSubcoreMesh(core_axis_name="c", subcore_axis_name="s")

    @functools.partial(
        pl.kernel,
        out_type=jax.ShapeDtypeStruct((NC, NS, DSTRIPE), jnp.float32),
        mesh=mesh,
        scratch_types=[
            pltpu.VMEM((KD, 128), jnp.int32),
            pltpu.VMEM((128,), jnp.float32),
            pltpu.VMEM_SHARED((DPAD,), jnp.float32),
        ],
    )
    def sc_degree(dst_hbm, zeros_hbm, ones_hbm, out_hbm, dst_v, ones_v, acc):
        c = lax.axis_index("c")
        s = lax.axis_index("s")
        wid = s * NC + c
        pltpu.sync_copy(dst_hbm.at[wid], dst_v)
        pltpu.sync_copy(ones_hbm, ones_v)
        pltpu.sync_copy(zeros_hbm, acc.at[pl.ds(s * DSTRIPE, DSTRIPE)])
        plsc.subcore_barrier()

        @pl.loop(0, KD)
        def _(j):
            pltpu.sync_copy(ones_v, acc.at[dst_v.at[j]], add=True)

        plsc.subcore_barrier()
        pltpu.sync_copy(acc.at[pl.ds(s * DSTRIPE, DSTRIPE)], out_hbm.at[c, s])

    @functools.partial(
        pl.kernel,
        out_type=jax.ShapeDtypeStruct((NC, NS, STRIPE, F), jnp.float32),
        mesh=mesh,
        scratch_types=[
            pltpu.VMEM((QMAX, C), jnp.int32),
            pltpu.VMEM((QMAX, C), jnp.int32),
            pltpu.VMEM((C, F), jnp.float32),
            pltpu.VMEM((C, F), jnp.float32),
            pltpu.VMEM_SHARED((NPAD, F), jnp.float32),
            pltpu.SemaphoreType.DMA,
            pltpu.SemaphoreType.DMA,
            pltpu.SemaphoreType.DMA,
            pltpu.SemaphoreType.DMA,
        ],
    )
    def sc_aggregate(g_hbm, src_hbm, dst_hbm, zeros_hbm, out_hbm,
                     src_v, dst_v, buf0, buf1, acc, sg0, sg1, ss0, ss1):
        c = lax.axis_index("c")
        s = lax.axis_index("s")
        pltpu.sync_copy(zeros_hbm, acc.at[pl.ds(s * STRIPE, STRIPE)])
        plsc.subcore_barrier()

        bufs = (buf0, buf1)
        sem_g = (sg0, sg1)
        sem_s = (ss0, ss1)

        def gathers(q, b):
            for u in range(U):
                pltpu.async_copy(
                    g_hbm.at[src_v.at[q, pl.ds(u * CU, CU)]],
                    bufs[b].at[pl.ds(u * CU, CU)], sem_g[b])

        def wait_gathers(b):
            for u in range(U):
                pltpu.make_async_copy(
                    g_hbm.at[src_v.at[0, pl.ds(0, CU)]],
                    bufs[b].at[pl.ds(0, CU)], sem_g[b]).wait()

        def scatter(q, b):
            pltpu.async_copy(bufs[b], acc.at[dst_v.at[q]], sem_s[b], add=True)

        def wait_scatter(b):
            pltpu.make_async_copy(bufs[b], acc.at[dst_v.at[0]],
                                  sem_s[b]).wait()

        def pipeline(base, phases):
            off = 0
            for Q in phases:
                pltpu.sync_copy(src_hbm.at[pl.ds(base + off, Q)],
                                src_v.at[pl.ds(0, Q)])
                pltpu.sync_copy(dst_hbm.at[pl.ds(base + off, Q)],
                                dst_v.at[pl.ds(0, Q)])
                off += Q
                gathers(0, 0)
                wait_gathers(0)
                scatter(0, 0)
                gathers(1, 1)
                @pl.loop(0, (Q - 2) // 2)
                def _(i):
                    for b in (1, 0):
                        q = 2 * i + 2 - b
                        wait_gathers(b)
                        scatter(q, b)
                        wait_scatter(1 - b)
                        gathers(q + 1, 1 - b)
                wait_gathers(1)
                scatter(Q - 1, 1)
                wait_scatter(0)
                wait_scatter(1)

        @pl.when(c == 1)
        def _():
            pipeline(s * RF, PH_F)

        @pl.when(c == 0)
        def _():
            pipeline(NS * RF + s * RS, PH_S)

        plsc.subcore_barrier()
        pltpu.sync_copy(acc.at[pl.ds(s * STRIPE, STRIPE)], out_hbm.at[c, s])

    return sc_degree, sc_aggregate


def _sc_degree(*args):
    return _sc_kernels()[0](*args)


def _sc_aggregate(*args):
    return _sc_kernels()[1](*args)


def _tc_prescale_kernel(d0_ref, d1_ref, x_ref, g_ref, dinv_ref, dinv2_ref):
    deg = d0_ref[...] + d1_ref[...] + 1.0
    dinv = lax.rsqrt(deg)
    dinv2 = 1.0 / deg
    dinv_ref[...] = dinv
    dinv2_ref[...] = dinv2
    g_ref[...] = x_ref[...] * dinv


def _tc_mid_kernel(a0_ref, a1_ref, x_ref, dinv_ref, dinv2_ref,
                   w1_ref, b1_ref, w2_ref, t2_ref, g2_ref):
    dinv = dinv_ref[...]
    p = dinv * (a0_ref[...] + a1_ref[...]) + dinv2_ref[...] * x_ref[...]
    h = jnp.maximum(
        jnp.dot(p, w1_ref[...], preferred_element_type=jnp.float32)
        + b1_ref[...], 0.0)
    t2 = jnp.dot(h, w2_ref[...], preferred_element_type=jnp.float32)
    t2_ref[...] = t2
    g2_ref[...] = t2 * dinv


def _tc_final_kernel(a0_ref, a1_ref, t2_ref, dinv_ref, dinv2_ref, b2_ref,
                     out_ref):
    o = (dinv_ref[...] * (a0_ref[...] + a1_ref[...])
         + dinv2_ref[...] * t2_ref[...] + b2_ref[...])
    m = jnp.max(o, axis=1, keepdims=True)
    e = jnp.exp(o - m)
    lse = jnp.log(jnp.sum(e, axis=1, keepdims=True)) + m
    out_ref[...] = o - lse


def _row_spec(width):
    return pl.BlockSpec((ROWB, width), lambda i: (i, 0))


def _full_spec(shape):
    return pl.BlockSpec(shape, lambda i: tuple(0 for _ in shape))


def kernel(x, edge_index, W1, b1, W2, b2):
    src = edge_index[0].astype(jnp.int32)
    dst = edge_index[1].astype(jnp.int32)
    pad = EPAD - E
    srcp = jnp.concatenate([src, jnp.zeros((pad,), jnp.int32)])
    dstp = jnp.concatenate([dst, jnp.full((pad,), N, jnp.int32)])
    src_r = srcp.reshape(NROWS, C)
    dst_r = dstp.reshape(NROWS, C)
    dst3 = dstp.reshape(NW, KD, C)

    xp = jnp.zeros((NPAD, F), jnp.float32).at[:N].set(x)
    zeros1 = jnp.zeros((DSTRIPE,), jnp.float32)
    zeros2 = jnp.zeros((STRIPE, F), jnp.float32)
    ones_c = jnp.ones((128,), jnp.float32)

    degp = _sc_degree(dst3, zeros1, ones_c).reshape(NC, DPAD)[:, :NPAD]
    d0 = degp[0].reshape(NPAD, 1)
    d1 = degp[1].reshape(NPAD, 1)

    g1, dinv, dinv2 = pl.pallas_call(
        _tc_prescale_kernel,
        grid=(GRID,),
        in_specs=[_row_spec(1), _row_spec(1), _row_spec(F)],
        out_specs=[_row_spec(F), _row_spec(1), _row_spec(1)],
        out_shape=[
            jax.ShapeDtypeStruct((NPAD, F), jnp.float32),
            jax.ShapeDtypeStruct((NPAD, 1), jnp.float32),
            jax.ShapeDtypeStruct((NPAD, 1), jnp.float32),
        ],
    )(d0, d1, xp)

    agg1 = _sc_aggregate(g1, src_r, dst_r, zeros2).reshape(NC, NPAD, F)

    t2, g2 = pl.pallas_call(
        _tc_mid_kernel,
        grid=(GRID,),
        in_specs=[
            _row_spec(F), _row_spec(F), _row_spec(F),
            _row_spec(1), _row_spec(1),
            _full_spec((F, HID)), _full_spec((1, HID)), _full_spec((HID, F)),
        ],
        out_specs=[_row_spec(F), _row_spec(F)],
        out_shape=[
            jax.ShapeDtypeStruct((NPAD, F), jnp.float32),
            jax.ShapeDtypeStruct((NPAD, F), jnp.float32),
        ],
    )(agg1[0], agg1[1], xp, dinv, dinv2, W1, b1.reshape(1, HID), W2)

    agg2 = _sc_aggregate(g2, src_r, dst_r, zeros2).reshape(NC, NPAD, F)

    out = pl.pallas_call(
        _tc_final_kernel,
        grid=(GRID,),
        in_specs=[
            _row_spec(F), _row_spec(F), _row_spec(F),
            _row_spec(1), _row_spec(1), _full_spec((1, F)),
        ],
        out_specs=_row_spec(F),
        out_shape=jax.ShapeDtypeStruct((NPAD, F), jnp.float32),
    )(agg2[0], agg2[1], t2, dinv, dinv2, b2.reshape(1, F))

    return out[:N]


# speedup vs baseline: 1.0558x; 1.0008x over previous
import functools

import jax
import jax.numpy as jnp
from jax import lax
from jax.experimental import pallas as pl
from jax.experimental.pallas import tpu as pltpu
from jax.experimental.pallas import tpu_sc as plsc

N = 10000
F = 128
HID = 256
E = 320000

NC = 2
NS = 16
NW = NC * NS

NPAD = 10112
STRIPE = NPAD // NS
C = 128
EPAD = 327680
KD = EPAD // NW // C
U = 4
CU = C // U
RF = 120
RS = 2 * KD - RF
PH_F = (64, 56)
PH_S = (24, 16)
QMAX = max(PH_F)
NROWS = EPAD // C
DPAD = 10240
DSTRIPE = DPAD // NS

ROWB = 632
GRID = NPAD // ROWB


@functools.cache
def _sc_kernels():
    mesh = plsc.VectorSubcoreMesh(core_axis_name="c", subcore_axis_name="s")

    @functools.partial(
        pl.kernel,
        out_type=jax.ShapeDtypeStruct((NC, NS, DSTRIPE), jnp.float32),
        mesh=mesh,
        scratch_types=[
            pltpu.VMEM((KD, 128), jnp.int32),
            pltpu.VMEM((128,), jnp.float32),
            pltpu.VMEM_SHARED((DPAD,), jnp.float32),
        ],
    )
    def sc_degree(dst_hbm, zeros_hbm, ones_hbm, out_hbm, dst_v, ones_v, acc):
        c = lax.axis_index("c")
        s = lax.axis_index("s")
        wid = s * NC + c
        pltpu.sync_copy(dst_hbm.at[wid], dst_v)
        pltpu.sync_copy(ones_hbm, ones_v)
        pltpu.sync_copy(zeros_hbm, acc.at[pl.ds(s * DSTRIPE, DSTRIPE)])
        plsc.subcore_barrier()

        @pl.loop(0, KD)
        def _(j):
            pltpu.sync_copy(ones_v, acc.at[dst_v.at[j]], add=True)

        plsc.subcore_barrier()
        pltpu.sync_copy(acc.at[pl.ds(s * DSTRIPE, DSTRIPE)], out_hbm.at[c, s])

    @functools.partial(
        pl.kernel,
        out_type=jax.ShapeDtypeStruct((NC, NS, STRIPE, F), jnp.float32),
        mesh=mesh,
        scratch_types=[
            pltpu.VMEM((QMAX, C), jnp.int32),
            pltpu.VMEM((QMAX, C), jnp.int32),
            pltpu.VMEM((C, F), jnp.float32),
            pltpu.VMEM((C, F), jnp.float32),
            pltpu.VMEM_SHARED((NPAD, F), jnp.float32),
            pltpu.SemaphoreType.DMA,
            pltpu.SemaphoreType.DMA,
            pltpu.SemaphoreType.DMA,
            pltpu.SemaphoreType.DMA,
        ],
    )
    def sc_aggregate(g_hbm, src_hbm, dst_hbm, zeros_hbm, out_hbm,
                     src_v, dst_v, buf0, buf1, acc, sg0, sg1, ss0, ss1):
        c = lax.axis_index("c")
        s = lax.axis_index("s")
        with jax.named_scope("agg_zero"):
            pltpu.sync_copy(zeros_hbm, acc.at[pl.ds(s * STRIPE, STRIPE)])
            plsc.subcore_barrier()

        bufs = (buf0, buf1)
        sem_g = (sg0, sg1)
        sem_s = (ss0, ss1)

        def gathers(q, b):
            for u in range(U):
                pltpu.async_copy(
                    g_hbm.at[src_v.at[q, pl.ds(u * CU, CU)]],
                    bufs[b].at[pl.ds(u * CU, CU)], sem_g[b])

        def wait_gathers(b):
            for u in range(U):
                pltpu.make_async_copy(
                    g_hbm.at[src_v.at[0, pl.ds(0, CU)]],
                    bufs[b].at[pl.ds(0, CU)], sem_g[b]).wait()

        def scatter(q, b):
            pltpu.async_copy(bufs[b], acc.at[dst_v.at[q]], sem_s[b], add=True)

        def wait_scatter(b):
            pltpu.make_async_copy(bufs[b], acc.at[dst_v.at[0]],
                                  sem_s[b]).wait()

        def pipeline(base, phases):
            off = 0
            for Q in phases:
                pltpu.sync_copy(src_hbm.at[pl.ds(base + off, Q)],
                                src_v.at[pl.ds(0, Q)])
                pltpu.sync_copy(dst_hbm.at[pl.ds(base + off, Q)],
                                dst_v.at[pl.ds(0, Q)])
                off += Q
                gathers(0, 0)
                wait_gathers(0)
                scatter(0, 0)
                gathers(1, 1)
                @pl.loop(0, (Q - 2) // 2)
                def _(i):
                    for b in (1, 0):
                        q = 2 * i + 2 - b
                        wait_gathers(b)
                        scatter(q, b)
                        wait_scatter(1 - b)
                        gathers(q + 1, 1 - b)
                wait_gathers(1)
                scatter(Q - 1, 1)
                wait_scatter(0)
                wait_scatter(1)

        with jax.named_scope("agg_edges"):
            @pl.when(c == 1)
            def _():
                pipeline(s * RF, PH_F)

            @pl.when(c == 0)
            def _():
                pipeline(NS * RF + s * RS, PH_S)

            plsc.subcore_barrier()
        with jax.named_scope("agg_out"):
            pltpu.sync_copy(acc.at[pl.ds(s * STRIPE, STRIPE)], out_hbm.at[c, s])

    return sc_degree, sc_aggregate


def _sc_degree(*args):
    return _sc_kernels()[0](*args)


def _sc_aggregate(*args):
    return _sc_kernels()[1](*args)


def _tc_prescale_kernel(d0_ref, d1_ref, x_ref, g_ref, dinv_ref, dinv2_ref):
    deg = d0_ref[...] + d1_ref[...] + 1.0
    dinv = lax.rsqrt(deg)
    dinv2 = 1.0 / deg
    dinv_ref[...] = dinv
    dinv2_ref[...] = dinv2
    g_ref[...] = x_ref[...] * dinv


def _tc_mid_kernel(a0_ref, a1_ref, x_ref, dinv_ref, dinv2_ref,
                   w1_ref, b1_ref, w2_ref, t2_ref, g2_ref):
    dinv = dinv_ref[...]
    p = dinv * (a0_ref[...] + a1_ref[...]) + dinv2_ref[...] * x_ref[...]
    h = jnp.maximum(
        jnp.dot(p, w1_ref[...], preferred_element_type=jnp.float32)
        + b1_ref[...], 0.0)
    t2 = jnp.dot(h, w2_ref[...], preferred_element_type=jnp.float32)
    t2_ref[...] = t2
    g2_ref[...] = t2 * dinv


def _tc_final_kernel(a0_ref, a1_ref, t2_ref, dinv_ref, dinv2_ref, b2_ref,
                     out_ref):
    o = (dinv_ref[...] * (a0_ref[...] + a1_ref[...])
         + dinv2_ref[...] * t2_ref[...] + b2_ref[...])
    m = jnp.max(o, axis=1, keepdims=True)
    e = jnp.exp(o - m)
    lse = jnp.log(jnp.sum(e, axis=1, keepdims=True)) + m
    out_ref[...] = o - lse


def _row_spec(width):
    return pl.BlockSpec((ROWB, width), lambda i: (i, 0))


def _full_spec(shape):
    return pl.BlockSpec(shape, lambda i: tuple(0 for _ in shape))


def kernel(x, edge_index, W1, b1, W2, b2):
    src = edge_index[0].astype(jnp.int32)
    dst = edge_index[1].astype(jnp.int32)
    pad = EPAD - E
    srcp = jnp.concatenate([src, jnp.zeros((pad,), jnp.int32)])
    dstp = jnp.concatenate([dst, jnp.full((pad,), N, jnp.int32)])
    src_r = srcp.reshape(NROWS, C)
    dst_r = dstp.reshape(NROWS, C)
    dst3 = dstp.reshape(NW, KD, C)

    xp = jnp.zeros((NPAD, F), jnp.float32).at[:N].set(x)
    zeros1 = jnp.zeros((DSTRIPE,), jnp.float32)
    zeros2 = jnp.zeros((STRIPE, F), jnp.float32)
    ones_c = jnp.ones((128,), jnp.float32)

    degp = _sc_degree(dst3, zeros1, ones_c).reshape(NC, DPAD)[:, :NPAD]
    d0 = degp[0].reshape(NPAD, 1)
    d1 = degp[1].reshape(NPAD, 1)

    g1, dinv, dinv2 = pl.pallas_call(
        _tc_prescale_kernel,
        grid=(GRID,),
        in_specs=[_row_spec(1), _row_spec(1), _row_spec(F)],
        out_specs=[_row_spec(F), _row_spec(1), _row_spec(1)],
        out_shape=[
            jax.ShapeDtypeStruct((NPAD, F), jnp.float32),
            jax.ShapeDtypeStruct((NPAD, 1), jnp.float32),
            jax.ShapeDtypeStruct((NPAD, 1), jnp.float32),
        ],
    )(d0, d1, xp)

    agg1 = _sc_aggregate(g1, src_r, dst_r, zeros2).reshape(NC, NPAD, F)

    t2, g2 = pl.pallas_call(
        _tc_mid_kernel,
        grid=(GRID,),
        in_specs=[
            _row_spec(F), _row_spec(F), _row_spec(F),
            _row_spec(1), _row_spec(1),
            _full_spec((F, HID)), _full_spec((1, HID)), _full_spec((HID, F)),
        ],
        out_specs=[_row_spec(F), _row_spec(F)],
        out_shape=[
            jax.ShapeDtypeStruct((NPAD, F), jnp.float32),
            jax.ShapeDtypeStruct((NPAD, F), jnp.float32),
        ],
    )(agg1[0], agg1[1], xp, dinv, dinv2, W1, b1.reshape(1, HID), W2)

    agg2 = _sc_aggregate(g2, src_r, dst_r, zeros2).reshape(NC, NPAD, F)

    out = pl.pallas_call(
        _tc_final_kernel,
        grid=(GRID,),
        in_specs=[
            _row_spec(F), _row_spec(F), _row_spec(F),
            _row_spec(1), _row_spec(1), _full_spec((1, F)),
        ],
        out_specs=_row_spec(F),
        out_shape=jax.ShapeDtypeStruct((NPAD, F), jnp.float32),
    )(agg2[0], agg2[1], t2, dinv, dinv2, b2.reshape(1, F))

    return out[:N]


# speedup vs baseline: 2.6390x; 2.4996x over previous
import functools

import jax
import jax.numpy as jnp
from jax import lax
from jax.experimental import pallas as pl
from jax.experimental.pallas import tpu as pltpu
from jax.experimental.pallas import tpu_sc as plsc

N = 10000
F = 128
HID = 256
E = 320000

NC = 2
NS = 16
NW = NC * NS

NPAD = 10112
STRIPE = NPAD // NS
C = 128
EPAD = 327680
KD = EPAD // NW // C
U = 4
CU = C // U
PH = (40, 40)
QMAX = max(PH)
NROWS = EPAD // C
DPAD = 10240
DSTRIPE = DPAD // NS

ROWB = 632
GRID = NPAD // ROWB


@functools.cache
def _sc_kernels():
    mesh = plsc.VectorSubcoreMesh(core_axis_name="c", subcore_axis_name="s")

    @functools.partial(
        pl.kernel,
        out_type=jax.ShapeDtypeStruct((NC, NS, DSTRIPE), jnp.float32),
        mesh=mesh,
        scratch_types=[
            pltpu.VMEM((KD, 128), jnp.int32),
            pltpu.VMEM((128,), jnp.float32),
            pltpu.VMEM_SHARED((DPAD,), jnp.float32),
        ],
    )
    def sc_degree(dst_hbm, zeros_hbm, ones_hbm, out_hbm, dst_v, ones_v, acc):
        c = lax.axis_index("c")
        s = lax.axis_index("s")
        wid = s * NC + c
        pltpu.sync_copy(dst_hbm.at[wid], dst_v)
        pltpu.sync_copy(ones_hbm, ones_v)
        pltpu.sync_copy(zeros_hbm, acc.at[pl.ds(s * DSTRIPE, DSTRIPE)])
        plsc.subcore_barrier()

        @pl.loop(0, KD)
        def _(j):
            pltpu.sync_copy(ones_v, acc.at[dst_v.at[j]], add=True)

        plsc.subcore_barrier()
        pltpu.sync_copy(acc.at[pl.ds(s * DSTRIPE, DSTRIPE)], out_hbm.at[c, s])

    @functools.partial(
        pl.kernel,
        out_type=jax.ShapeDtypeStruct((NC, NS, STRIPE, F), jnp.float32),
        mesh=mesh,
        scratch_types=[
            pltpu.VMEM((QMAX, C), jnp.int32),
            pltpu.VMEM((QMAX, C), jnp.int32),
            pltpu.VMEM((C, F), jnp.float32),
            pltpu.VMEM((C, F), jnp.float32),
            pltpu.VMEM_SHARED((NPAD, F), jnp.float32),
            pltpu.SemaphoreType.DMA,
            pltpu.SemaphoreType.DMA,
            pltpu.SemaphoreType.DMA,
            pltpu.SemaphoreType.DMA,
        ],
    )
    def sc_aggregate(g_hbm, src_hbm, dst_hbm, zeros_hbm, out_hbm,
                     src_v, dst_v, buf0, buf1, acc, sg0, sg1, ss0, ss1):
        c = lax.axis_index("c")
        s = lax.axis_index("s")
        with jax.named_scope("agg_zero"):
            pltpu.sync_copy(zeros_hbm, acc.at[pl.ds(s * STRIPE, STRIPE)])
            plsc.subcore_barrier()

        bufs = (buf0, buf1)
        sem_g = (sg0, sg1)
        sem_s = (ss0, ss1)

        def gathers(q, b):
            for u in range(U):
                pltpu.async_copy(
                    g_hbm.at[src_v.at[q, pl.ds(u * CU, CU)]],
                    bufs[b].at[pl.ds(u * CU, CU)], sem_g[b])

        def wait_gathers(b):
            for u in range(U):
                pltpu.make_async_copy(
                    g_hbm.at[src_v.at[0, pl.ds(0, CU)]],
                    bufs[b].at[pl.ds(0, CU)], sem_g[b]).wait()

        def scatter(q, b):
            pltpu.async_copy(bufs[b], acc.at[dst_v.at[q]], sem_s[b], add=True)

        def wait_scatter(b):
            pltpu.make_async_copy(bufs[b], acc.at[dst_v.at[0]],
                                  sem_s[b]).wait()

        def pipeline(base, phases):
            off = 0
            for Q in phases:
                pltpu.sync_copy(src_hbm.at[pl.ds(base + off, Q)],
                                src_v.at[pl.ds(0, Q)])
                pltpu.sync_copy(dst_hbm.at[pl.ds(base + off, Q)],
                                dst_v.at[pl.ds(0, Q)])
                off += Q
                gathers(0, 0)
                wait_gathers(0)
                scatter(0, 0)
                gathers(1, 1)
                @pl.loop(0, (Q - 2) // 2)
                def _(i):
                    for b in (1, 0):
                        q = 2 * i + 2 - b
                        wait_gathers(b)
                        scatter(q, b)
                        wait_scatter(1 - b)
                        gathers(q + 1, 1 - b)
                wait_gathers(1)
                scatter(Q - 1, 1)
                wait_scatter(0)
                wait_scatter(1)

        with jax.named_scope("agg_edges"):
            pipeline((s * NC + c) * KD, PH)
            plsc.subcore_barrier()
        with jax.named_scope("agg_out"):
            pltpu.sync_copy(acc.at[pl.ds(s * STRIPE, STRIPE)], out_hbm.at[c, s])

    return sc_degree, sc_aggregate


def _sc_degree(*args):
    return _sc_kernels()[0](*args)


def _sc_aggregate(*args):
    return _sc_kernels()[1](*args)


def _tc_prescale_kernel(d0_ref, d1_ref, x_ref, g_ref, dinv_ref, dinv2_ref):
    deg = d0_ref[...] + d1_ref[...] + 1.0
    dinv = lax.rsqrt(deg)
    dinv2 = 1.0 / deg
    dinv_ref[...] = dinv
    dinv2_ref[...] = dinv2
    g_ref[...] = x_ref[...] * dinv


def _tc_mid_kernel(a0_ref, a1_ref, x_ref, dinv_ref, dinv2_ref,
                   w1_ref, b1_ref, w2_ref, t2_ref, g2_ref):
    dinv = dinv_ref[...]
    p = dinv * (a0_ref[...] + a1_ref[...]) + dinv2_ref[...] * x_ref[...]
    h = jnp.maximum(
        jnp.dot(p, w1_ref[...], preferred_element_type=jnp.float32)
        + b1_ref[...], 0.0)
    t2 = jnp.dot(h, w2_ref[...], preferred_element_type=jnp.float32)
    t2_ref[...] = t2
    g2_ref[...] = t2 * dinv


def _tc_final_kernel(a0_ref, a1_ref, t2_ref, dinv_ref, dinv2_ref, b2_ref,
                     out_ref):
    o = (dinv_ref[...] * (a0_ref[...] + a1_ref[...])
         + dinv2_ref[...] * t2_ref[...] + b2_ref[...])
    m = jnp.max(o, axis=1, keepdims=True)
    e = jnp.exp(o - m)
    lse = jnp.log(jnp.sum(e, axis=1, keepdims=True)) + m
    out_ref[...] = o - lse


def _row_spec(width):
    return pl.BlockSpec((ROWB, width), lambda i: (i, 0))


def _full_spec(shape):
    return pl.BlockSpec(shape, lambda i: tuple(0 for _ in shape))


def kernel(x, edge_index, W1, b1, W2, b2):
    src = edge_index[0].astype(jnp.int32)
    dst = edge_index[1].astype(jnp.int32)
    pad = EPAD - E
    pidx = jnp.arange(pad, dtype=jnp.int32)
    srcp = jnp.concatenate([src, pidx % N])
    dstp = jnp.concatenate([dst, N + pidx % (NPAD - N)])
    src_r = srcp.reshape(NROWS, C)
    dst_r = dstp.reshape(NROWS, C)
    dst3 = dstp.reshape(NW, KD, C)

    xp = jnp.zeros((NPAD, F), jnp.float32).at[:N].set(x)
    zeros1 = jnp.zeros((DSTRIPE,), jnp.float32)
    zeros2 = jnp.zeros((STRIPE, F), jnp.float32)
    ones_c = jnp.ones((128,), jnp.float32)

    degp = _sc_degree(dst3, zeros1, ones_c).reshape(NC, DPAD)[:, :NPAD]
    d0 = degp[0].reshape(NPAD, 1)
    d1 = degp[1].reshape(NPAD, 1)

    g1, dinv, dinv2 = pl.pallas_call(
        _tc_prescale_kernel,
        grid=(GRID,),
        in_specs=[_row_spec(1), _row_spec(1), _row_spec(F)],
        out_specs=[_row_spec(F), _row_spec(1), _row_spec(1)],
        out_shape=[
            jax.ShapeDtypeStruct((NPAD, F), jnp.float32),
            jax.ShapeDtypeStruct((NPAD, 1), jnp.float32),
            jax.ShapeDtypeStruct((NPAD, 1), jnp.float32),
        ],
    )(d0, d1, xp)

    agg1 = _sc_aggregate(g1, src_r, dst_r, zeros2).reshape(NC, NPAD, F)

    t2, g2 = pl.pallas_call(
        _tc_mid_kernel,
        grid=(GRID,),
        in_specs=[
            _row_spec(F), _row_spec(F), _row_spec(F),
            _row_spec(1), _row_spec(1),
            _full_spec((F, HID)), _full_spec((1, HID)), _full_spec((HID, F)),
        ],
        out_specs=[_row_spec(F), _row_spec(F)],
        out_shape=[
            jax.ShapeDtypeStruct((NPAD, F), jnp.float32),
            jax.ShapeDtypeStruct((NPAD, F), jnp.float32),
        ],
    )(agg1[0], agg1[1], xp, dinv, dinv2, W1, b1.reshape(1, HID), W2)

    agg2 = _sc_aggregate(g2, src_r, dst_r, zeros2).reshape(NC, NPAD, F)

    out = pl.pallas_call(
        _tc_final_kernel,
        grid=(GRID,),
        in_specs=[
            _row_spec(F), _row_spec(F), _row_spec(F),
            _row_spec(1), _row_spec(1), _full_spec((1, F)),
        ],
        out_specs=_row_spec(F),
        out_shape=jax.ShapeDtypeStruct((NPAD, F), jnp.float32),
    )(agg2[0], agg2[1], t2, dinv, dinv2, b2.reshape(1, F))

    return out[:N]


# speedup vs baseline: 2.9448x; 1.1159x over previous
import functools

import numpy as np

import jax
import jax.numpy as jnp
from jax import lax
from jax.experimental import pallas as pl
from jax.experimental.pallas import tpu as pltpu
from jax.experimental.pallas import tpu_sc as plsc

N = 10000
F = 128
HID = 256
E = 320000

NC = 2
NS = 16
NW = NC * NS

NPAD = 10112
STRIPE = NPAD // NS
C = 128
EPAD = 327680
KD = EPAD // NW // C
U = 4
CU = C // U
PH = (40, 40)
QMAX = max(PH)
NROWS = EPAD // C
DPAD = 10240
DSTRIPE = DPAD // NS

ROWB = 2000
GRID = N // ROWB


@functools.cache
def _sc_kernels():
    mesh = plsc.VectorSubcoreMesh(core_axis_name="c", subcore_axis_name="s")

    @functools.partial(
        pl.kernel,
        out_type=jax.ShapeDtypeStruct((NC, DPAD), jnp.float32),
        mesh=mesh,
        scratch_types=[
            pltpu.VMEM((KD, 128), jnp.int32),
            pltpu.VMEM((128,), jnp.float32),
            pltpu.VMEM_SHARED((DPAD,), jnp.float32),
        ],
    )
    def sc_degree(dst_hbm, zeros_hbm, ones_hbm, out_hbm, dst_v, ones_v, acc):
        c = lax.axis_index("c")
        s = lax.axis_index("s")
        wid = s * NC + c
        pltpu.sync_copy(dst_hbm.at[wid], dst_v)
        pltpu.sync_copy(ones_hbm, ones_v)
        pltpu.sync_copy(zeros_hbm, acc.at[pl.ds(s * DSTRIPE, DSTRIPE)])
        plsc.subcore_barrier()

        @pl.loop(0, KD)
        def _(j):
            pltpu.sync_copy(ones_v, acc.at[dst_v.at[j]], add=True)

        plsc.subcore_barrier()
        pltpu.sync_copy(acc.at[pl.ds(s * DSTRIPE, DSTRIPE)],
                        out_hbm.at[c, pl.ds(s * DSTRIPE, DSTRIPE)])

    @functools.partial(
        pl.kernel,
        out_type=jax.ShapeDtypeStruct((NC, NPAD, F), jnp.float32),
        mesh=mesh,
        scratch_types=[
            pltpu.VMEM((QMAX, C), jnp.int32),
            pltpu.VMEM((QMAX, C), jnp.int32),
            pltpu.VMEM((C, F), jnp.float32),
            pltpu.VMEM((C, F), jnp.float32),
            pltpu.VMEM_SHARED((NPAD, F), jnp.float32),
            pltpu.SemaphoreType.DMA,
            pltpu.SemaphoreType.DMA,
            pltpu.SemaphoreType.DMA,
            pltpu.SemaphoreType.DMA,
        ],
    )
    def sc_aggregate(g_hbm, src_hbm, dst_hbm, zeros_hbm, out_hbm,
                     src_v, dst_v, buf0, buf1, acc, sg0, sg1, ss0, ss1):
        c = lax.axis_index("c")
        s = lax.axis_index("s")
        with jax.named_scope("agg_zero"):
            pltpu.sync_copy(zeros_hbm, acc.at[pl.ds(s * STRIPE, STRIPE)])
            plsc.subcore_barrier()

        bufs = (buf0, buf1)
        sem_g = (sg0, sg1)
        sem_s = (ss0, ss1)

        def gathers(q, b):
            for u in range(U):
                pltpu.async_copy(
                    g_hbm.at[src_v.at[q, pl.ds(u * CU, CU)]],
                    bufs[b].at[pl.ds(u * CU, CU)], sem_g[b])

        def wait_gathers(b):
            for u in range(U):
                pltpu.make_async_copy(
                    g_hbm.at[src_v.at[0, pl.ds(0, CU)]],
                    bufs[b].at[pl.ds(0, CU)], sem_g[b]).wait()

        def scatter(q, b):
            pltpu.async_copy(bufs[b], acc.at[dst_v.at[q]], sem_s[b], add=True)

        def wait_scatter(b):
            pltpu.make_async_copy(bufs[b], acc.at[dst_v.at[0]],
                                  sem_s[b]).wait()

        def pipeline(base, phases):
            off = 0
            for Q in phases:
                pltpu.sync_copy(src_hbm.at[pl.ds(base + off, Q)],
                                src_v.at[pl.ds(0, Q)])
                pltpu.sync_copy(dst_hbm.at[pl.ds(base + off, Q)],
                                dst_v.at[pl.ds(0, Q)])
                off += Q
                gathers(0, 0)
                wait_gathers(0)
                scatter(0, 0)
                gathers(1, 1)
                @pl.loop(0, (Q - 2) // 2)
                def _(i):
                    for b in (1, 0):
                        q = 2 * i + 2 - b
                        wait_gathers(b)
                        scatter(q, b)
                        wait_scatter(1 - b)
                        gathers(q + 1, 1 - b)
                wait_gathers(1)
                scatter(Q - 1, 1)
                wait_scatter(0)
                wait_scatter(1)

        with jax.named_scope("agg_edges"):
            pipeline((s * NC + c) * KD, PH)
            plsc.subcore_barrier()
        with jax.named_scope("agg_out"):
            pltpu.sync_copy(acc.at[pl.ds(s * STRIPE, STRIPE)],
                            out_hbm.at[c, pl.ds(s * STRIPE, STRIPE)])

    return sc_degree, sc_aggregate


def _sc_degree(*args):
    return _sc_kernels()[0](*args)


def _sc_aggregate(*args):
    return _sc_kernels()[1](*args)


def _tc_prescale_kernel(d0_ref, d1_ref, x_ref, g_ref, dinv_ref, dinv2_ref):
    deg = d0_ref[...] + d1_ref[...] + 1.0
    dinv = lax.rsqrt(deg)
    dinv2 = 1.0 / deg
    dinv_ref[...] = dinv
    dinv2_ref[...] = dinv2
    g_ref[...] = x_ref[...] * dinv


def _tc_mid_kernel(a0_ref, a1_ref, x_ref, dinv_ref, dinv2_ref,
                   w1_ref, b1_ref, w2_ref, t2_ref, g2_ref):
    dinv = dinv_ref[...]
    p = dinv * (a0_ref[0] + a1_ref[0]) + dinv2_ref[...] * x_ref[...]
    h = jnp.maximum(
        jnp.dot(p, w1_ref[...], preferred_element_type=jnp.float32)
        + b1_ref[...], 0.0)
    t2 = jnp.dot(h, w2_ref[...], preferred_element_type=jnp.float32)
    t2_ref[...] = t2
    g2_ref[...] = t2 * dinv


def _tc_final_kernel(a0_ref, a1_ref, t2_ref, dinv_ref, dinv2_ref, b2_ref,
                     out_ref):
    o = (dinv_ref[...] * (a0_ref[0] + a1_ref[0])
         + dinv2_ref[...] * t2_ref[...] + b2_ref[...])
    m = jnp.max(o, axis=1, keepdims=True)
    e = jnp.exp(o - m)
    lse = jnp.log(jnp.sum(e, axis=1, keepdims=True)) + m
    out_ref[...] = o - lse


def _row_spec(width):
    return pl.BlockSpec((ROWB, width), lambda i: (i, 0))


def _agg_spec(core):
    return pl.BlockSpec((1, ROWB, F), lambda i, core=core: (core, i, 0))


def _full_spec(shape):
    return pl.BlockSpec(shape, lambda i: tuple(0 for _ in shape))


_PAD = EPAD - E
_PAD_SRC = np.arange(_PAD, dtype=np.int32) % N
_PAD_DST = N + np.arange(_PAD, dtype=np.int32) % (NPAD - N)


def kernel(x, edge_index, W1, b1, W2, b2):
    src = edge_index[0].astype(jnp.int32)
    dst = edge_index[1].astype(jnp.int32)
    srcp = jnp.concatenate([src, jnp.asarray(_PAD_SRC)])
    dstp = jnp.concatenate([dst, jnp.asarray(_PAD_DST)])
    src_r = srcp.reshape(NROWS, C)
    dst_r = dstp.reshape(NROWS, C)
    dst3 = dstp.reshape(NW, KD, C)

    zeros1 = jnp.zeros((DSTRIPE,), jnp.float32)
    zeros2 = jnp.zeros((STRIPE, F), jnp.float32)
    ones_c = jnp.ones((128,), jnp.float32)

    degp = _sc_degree(dst3, zeros1, ones_c)
    d0 = degp[0, :N].reshape(N, 1)
    d1 = degp[1, :N].reshape(N, 1)

    g1, dinv, dinv2 = pl.pallas_call(
        _tc_prescale_kernel,
        grid=(GRID,),
        in_specs=[_row_spec(1), _row_spec(1), _row_spec(F)],
        out_specs=[_row_spec(F), _row_spec(1), _row_spec(1)],
        out_shape=[
            jax.ShapeDtypeStruct((N, F), jnp.float32),
            jax.ShapeDtypeStruct((N, 1), jnp.float32),
            jax.ShapeDtypeStruct((N, 1), jnp.float32),
        ],
    )(d0, d1, x)

    agg1 = _sc_aggregate(g1, src_r, dst_r, zeros2)

    t2, g2 = pl.pallas_call(
        _tc_mid_kernel,
        grid=(GRID,),
        in_specs=[
            _agg_spec(0), _agg_spec(1), _row_spec(F),
            _row_spec(1), _row_spec(1),
            _full_spec((F, HID)), _full_spec((1, HID)), _full_spec((HID, F)),
        ],
        out_specs=[_row_spec(F), _row_spec(F)],
        out_shape=[
            jax.ShapeDtypeStruct((N, F), jnp.float32),
            jax.ShapeDtypeStruct((N, F), jnp.float32),
        ],
    )(agg1, agg1, x, dinv, dinv2, W1, b1.reshape(1, HID), W2)

    agg2 = _sc_aggregate(g2, src_r, dst_r, zeros2)

    out = pl.pallas_call(
        _tc_final_kernel,
        grid=(GRID,),
        in_specs=[
            _agg_spec(0), _agg_spec(1), _row_spec(F),
            _row_spec(1), _row_spec(1), _full_spec((1, F)),
        ],
        out_specs=_row_spec(F),
        out_shape=jax.ShapeDtypeStruct((N, F), jnp.float32),
    )(agg2, agg2, t2, dinv, dinv2, b2.reshape(1, F))

    return out


# speedup vs baseline: 2.9556x; 1.0037x over previous
import functools

import numpy as np

import jax
import jax.numpy as jnp
from jax import lax
from jax.experimental import pallas as pl
from jax.experimental.pallas import tpu as pltpu
from jax.experimental.pallas import tpu_sc as plsc

N = 10000
F = 128
HID = 256
E = 320000

NC = 2
NS = 16
NW = NC * NS

NPAD = 10112
STRIPE = NPAD // NS
C = 128
EPAD = 327680
KD = EPAD // NW // C
U = 1
CU = C // U
PH = (40, 40)
QMAX = max(PH)
NROWS = EPAD // C
DPAD = 10240
DSTRIPE = DPAD // NS

ROWB = 2000
GRID = N // ROWB


@functools.cache
def _sc_kernels():
    mesh = plsc.VectorSubcoreMesh(core_axis_name="c", subcore_axis_name="s")

    @functools.partial(
        pl.kernel,
        out_type=jax.ShapeDtypeStruct((NC, DPAD), jnp.float32),
        mesh=mesh,
        scratch_types=[
            pltpu.VMEM((KD, 128), jnp.int32),
            pltpu.VMEM((128,), jnp.float32),
            pltpu.VMEM_SHARED((DPAD,), jnp.float32),
        ],
    )
    def sc_degree(dst_hbm, zeros_hbm, ones_hbm, out_hbm, dst_v, ones_v, acc):
        c = lax.axis_index("c")
        s = lax.axis_index("s")
        wid = s * NC + c
        pltpu.sync_copy(dst_hbm.at[wid], dst_v)
        pltpu.sync_copy(ones_hbm, ones_v)
        pltpu.sync_copy(zeros_hbm, acc.at[pl.ds(s * DSTRIPE, DSTRIPE)])
        plsc.subcore_barrier()

        @pl.loop(0, KD)
        def _(j):
            pltpu.sync_copy(ones_v, acc.at[dst_v.at[j]], add=True)

        plsc.subcore_barrier()
        pltpu.sync_copy(acc.at[pl.ds(s * DSTRIPE, DSTRIPE)],
                        out_hbm.at[c, pl.ds(s * DSTRIPE, DSTRIPE)])

    @functools.partial(
        pl.kernel,
        out_type=jax.ShapeDtypeStruct((NC, NPAD, F), jnp.float32),
        mesh=mesh,
        scratch_types=[
            pltpu.VMEM((QMAX, C), jnp.int32),
            pltpu.VMEM((QMAX, C), jnp.int32),
            pltpu.VMEM((C, F), jnp.float32),
            pltpu.VMEM((C, F), jnp.float32),
            pltpu.VMEM_SHARED((NPAD, F), jnp.float32),
            pltpu.SemaphoreType.DMA,
            pltpu.SemaphoreType.DMA,
            pltpu.SemaphoreType.DMA,
            pltpu.SemaphoreType.DMA,
        ],
    )
    def sc_aggregate(g_hbm, src_hbm, dst_hbm, zeros_hbm, out_hbm,
                     src_v, dst_v, buf0, buf1, acc, sg0, sg1, ss0, ss1):
        c = lax.axis_index("c")
        s = lax.axis_index("s")
        with jax.named_scope("agg_zero"):
            pltpu.sync_copy(zeros_hbm, acc.at[pl.ds(s * STRIPE, STRIPE)])
            plsc.subcore_barrier()

        bufs = (buf0, buf1)
        sem_g = (sg0, sg1)
        sem_s = (ss0, ss1)

        def gathers(q, b):
            for u in range(U):
                pltpu.async_copy(
                    g_hbm.at[src_v.at[q, pl.ds(u * CU, CU)]],
                    bufs[b].at[pl.ds(u * CU, CU)], sem_g[b])

        def wait_gathers(b):
            for u in range(U):
                pltpu.make_async_copy(
                    g_hbm.at[src_v.at[0, pl.ds(0, CU)]],
                    bufs[b].at[pl.ds(0, CU)], sem_g[b]).wait()

        def scatter(q, b):
            pltpu.async_copy(bufs[b], acc.at[dst_v.at[q]], sem_s[b], add=True)

        def wait_scatter(b):
            pltpu.make_async_copy(bufs[b], acc.at[dst_v.at[0]],
                                  sem_s[b]).wait()

        def pipeline(base, phases):
            off = 0
            for Q in phases:
                pltpu.sync_copy(src_hbm.at[pl.ds(base + off, Q)],
                                src_v.at[pl.ds(0, Q)])
                pltpu.sync_copy(dst_hbm.at[pl.ds(base + off, Q)],
                                dst_v.at[pl.ds(0, Q)])
                off += Q
                gathers(0, 0)
                wait_gathers(0)
                scatter(0, 0)
                gathers(1, 1)
                @pl.loop(0, (Q - 2) // 2)
                def _(i):
                    for b in (1, 0):
                        q = 2 * i + 2 - b
                        wait_gathers(b)
                        scatter(q, b)
                        wait_scatter(1 - b)
                        gathers(q + 1, 1 - b)
                wait_gathers(1)
                scatter(Q - 1, 1)
                wait_scatter(0)
                wait_scatter(1)

        with jax.named_scope("agg_edges"):
            pipeline((s * NC + c) * KD, PH)
            plsc.subcore_barrier()
        with jax.named_scope("agg_out"):
            pltpu.sync_copy(acc.at[pl.ds(s * STRIPE, STRIPE)],
                            out_hbm.at[c, pl.ds(s * STRIPE, STRIPE)])

    return sc_degree, sc_aggregate


def _sc_degree(*args):
    return _sc_kernels()[0](*args)


def _sc_aggregate(*args):
    return _sc_kernels()[1](*args)


def _tc_prescale_kernel(d0_ref, d1_ref, x_ref, g_ref, dinv_ref, dinv2_ref):
    deg = d0_ref[...] + d1_ref[...] + 1.0
    dinv = lax.rsqrt(deg)
    dinv2 = 1.0 / deg
    dinv_ref[...] = dinv
    dinv2_ref[...] = dinv2
    g_ref[...] = x_ref[...] * dinv


def _tc_mid_kernel(a0_ref, a1_ref, x_ref, dinv_ref, dinv2_ref,
                   w1_ref, b1_ref, w2_ref, t2_ref, g2_ref):
    dinv = dinv_ref[...]
    p = dinv * (a0_ref[0] + a1_ref[0]) + dinv2_ref[...] * x_ref[...]
    h = jnp.maximum(
        jnp.dot(p, w1_ref[...], preferred_element_type=jnp.float32)
        + b1_ref[...], 0.0)
    t2 = jnp.dot(h, w2_ref[...], preferred_element_type=jnp.float32)
    t2_ref[...] = t2
    g2_ref[...] = t2 * dinv


def _tc_final_kernel(a0_ref, a1_ref, t2_ref, dinv_ref, dinv2_ref, b2_ref,
                     out_ref):
    o = (dinv_ref[...] * (a0_ref[0] + a1_ref[0])
         + dinv2_ref[...] * t2_ref[...] + b2_ref[...])
    m = jnp.max(o, axis=1, keepdims=True)
    e = jnp.exp(o - m)
    lse = jnp.log(jnp.sum(e, axis=1, keepdims=True)) + m
    out_ref[...] = o - lse


def _row_spec(width):
    return pl.BlockSpec((ROWB, width), lambda i: (i, 0))


def _agg_spec(core):
    return pl.BlockSpec((1, ROWB, F), lambda i, core=core: (core, i, 0))


def _full_spec(shape):
    return pl.BlockSpec(shape, lambda i: tuple(0 for _ in shape))


_PAD = EPAD - E
_PAD_SRC = np.arange(_PAD, dtype=np.int32) % N
_PAD_DST = N + np.arange(_PAD, dtype=np.int32) % (NPAD - N)


def kernel(x, edge_index, W1, b1, W2, b2):
    src = edge_index[0].astype(jnp.int32)
    dst = edge_index[1].astype(jnp.int32)
    srcp = jnp.concatenate([src, jnp.asarray(_PAD_SRC)])
    dstp = jnp.concatenate([dst, jnp.asarray(_PAD_DST)])
    src_r = srcp.reshape(NROWS, C)
    dst_r = dstp.reshape(NROWS, C)
    dst3 = dstp.reshape(NW, KD, C)

    zeros1 = jnp.zeros((DSTRIPE,), jnp.float32)
    zeros2 = jnp.zeros((STRIPE, F), jnp.float32)
    ones_c = jnp.ones((128,), jnp.float32)

    degp = _sc_degree(dst3, zeros1, ones_c)
    d0 = degp[0, :N].reshape(N, 1)
    d1 = degp[1, :N].reshape(N, 1)

    g1, dinv, dinv2 = pl.pallas_call(
        _tc_prescale_kernel,
        grid=(GRID,),
        in_specs=[_row_spec(1), _row_spec(1), _row_spec(F)],
        out_specs=[_row_spec(F), _row_spec(1), _row_spec(1)],
        out_shape=[
            jax.ShapeDtypeStruct((N, F), jnp.float32),
            jax.ShapeDtypeStruct((N, 1), jnp.float32),
            jax.ShapeDtypeStruct((N, 1), jnp.float32),
        ],
    )(d0, d1, x)

    agg1 = _sc_aggregate(g1, src_r, dst_r, zeros2)

    t2, g2 = pl.pallas_call(
        _tc_mid_kernel,
        grid=(GRID,),
        in_specs=[
            _agg_spec(0), _agg_spec(1), _row_spec(F),
            _row_spec(1), _row_spec(1),
            _full_spec((F, HID)), _full_spec((1, HID)), _full_spec((HID, F)),
        ],
        out_specs=[_row_spec(F), _row_spec(F)],
        out_shape=[
            jax.ShapeDtypeStruct((N, F), jnp.float32),
            jax.ShapeDtypeStruct((N, F), jnp.float32),
        ],
    )(agg1, agg1, x, dinv, dinv2, W1, b1.reshape(1, HID), W2)

    agg2 = _sc_aggregate(g2, src_r, dst_r, zeros2)

    out = pl.pallas_call(
        _tc_final_kernel,
        grid=(GRID,),
        in_specs=[
            _agg_spec(0), _agg_spec(1), _row_spec(F),
            _row_spec(1), _row_spec(1), _full_spec((1, F)),
        ],
        out_specs=_row_spec(F),
        out_shape=jax.ShapeDtypeStruct((N, F), jnp.float32),
    )(agg2, agg2, t2, dinv, dinv2, b2.reshape(1, F))

    return out


# speedup vs baseline: 3.0673x; 1.0378x over previous
import functools

import numpy as np

import jax
import jax.numpy as jnp
from jax import lax
from jax.experimental import pallas as pl
from jax.experimental.pallas import tpu as pltpu
from jax.experimental.pallas import tpu_sc as plsc

N = 10000
F = 128
HID = 256
E = 320000

NC = 2
NS = 16
NW = NC * NS

NPAD = 10112
STRIPE = NPAD // NS
C = 128
EPAD = 327680
KD = EPAD // NW // C
U = 1
CU = C // U
PH = (40, 40)
QMAX = max(PH)
NROWS = EPAD // C
DPAD = 10240
DSTRIPE = DPAD // NS

ROWB = 2000
GRID = N // ROWB


@functools.cache
def _sc_kernels():
    mesh = plsc.VectorSubcoreMesh(core_axis_name="c", subcore_axis_name="s")

    @functools.partial(
        pl.kernel,
        out_type=jax.ShapeDtypeStruct((NC, DPAD), jnp.float32),
        mesh=mesh,
        scratch_types=[
            pltpu.VMEM((KD, 2, 128), jnp.int32),
            pltpu.VMEM((128,), jnp.float32),
            pltpu.VMEM_SHARED((DPAD,), jnp.float32),
        ],
    )
    def sc_degree(edge_hbm, zeros_hbm, ones_hbm, out_hbm, dst_v, ones_v, acc):
        c = lax.axis_index("c")
        s = lax.axis_index("s")
        wid = s * NC + c
        pltpu.sync_copy(edge_hbm.at[pl.ds(wid * KD, KD)], dst_v)
        pltpu.sync_copy(ones_hbm, ones_v)
        pltpu.sync_copy(zeros_hbm, acc.at[pl.ds(s * DSTRIPE, DSTRIPE)])
        plsc.subcore_barrier()

        @pl.loop(0, KD)
        def _(j):
            pltpu.sync_copy(ones_v, acc.at[dst_v.at[j, 1]], add=True)

        plsc.subcore_barrier()
        pltpu.sync_copy(acc.at[pl.ds(s * DSTRIPE, DSTRIPE)],
                        out_hbm.at[c, pl.ds(s * DSTRIPE, DSTRIPE)])

    @functools.partial(
        pl.kernel,
        out_type=jax.ShapeDtypeStruct((NC, NPAD, F), jnp.float32),
        mesh=mesh,
        scratch_types=[
            pltpu.VMEM((QMAX, 2, C), jnp.int32),
            pltpu.VMEM((C, F), jnp.float32),
            pltpu.VMEM((C, F), jnp.float32),
            pltpu.VMEM_SHARED((NPAD, F), jnp.float32),
            pltpu.SemaphoreType.DMA,
            pltpu.SemaphoreType.DMA,
            pltpu.SemaphoreType.DMA,
            pltpu.SemaphoreType.DMA,
        ],
    )
    def sc_aggregate(g_hbm, edge_hbm, zeros_hbm, out_hbm,
                     idx_v, buf0, buf1, acc, sg0, sg1, ss0, ss1):
        c = lax.axis_index("c")
        s = lax.axis_index("s")
        with jax.named_scope("agg_zero"):
            pltpu.sync_copy(zeros_hbm, acc.at[pl.ds(s * STRIPE, STRIPE)])
            plsc.subcore_barrier()

        bufs = (buf0, buf1)
        sem_g = (sg0, sg1)
        sem_s = (ss0, ss1)

        def gathers(q, b):
            pltpu.async_copy(g_hbm.at[idx_v.at[q, 0]], bufs[b], sem_g[b])

        def wait_gathers(b):
            pltpu.make_async_copy(g_hbm.at[idx_v.at[0, 0]], bufs[b],
                                  sem_g[b]).wait()

        def scatter(q, b):
            pltpu.async_copy(bufs[b], acc.at[idx_v.at[q, 1]], sem_s[b],
                             add=True)

        def wait_scatter(b):
            pltpu.make_async_copy(bufs[b], acc.at[idx_v.at[0, 1]],
                                  sem_s[b]).wait()

        def pipeline(base, phases):
            off = 0
            for Q in phases:
                pltpu.sync_copy(edge_hbm.at[pl.ds(base + off, Q)],
                                idx_v.at[pl.ds(0, Q)])
                off += Q
                gathers(0, 0)
                wait_gathers(0)
                scatter(0, 0)
                gathers(1, 1)
                @pl.loop(0, (Q - 2) // 2)
                def _(i):
                    for b in (1, 0):
                        q = 2 * i + 2 - b
                        wait_gathers(b)
                        scatter(q, b)
                        wait_scatter(1 - b)
                        gathers(q + 1, 1 - b)
                wait_gathers(1)
                scatter(Q - 1, 1)
                wait_scatter(0)
                wait_scatter(1)

        with jax.named_scope("agg_edges"):
            pipeline((s * NC + c) * KD, PH)
            plsc.subcore_barrier()
        with jax.named_scope("agg_out"):
            pltpu.sync_copy(acc.at[pl.ds(s * STRIPE, STRIPE)],
                            out_hbm.at[c, pl.ds(s * STRIPE, STRIPE)])

    return sc_degree, sc_aggregate


def _sc_degree(*args):
    return _sc_kernels()[0](*args)


def _sc_aggregate(*args):
    return _sc_kernels()[1](*args)


def _dinvs(d0_ref, d1_ref):
    deg = d0_ref[...] + d1_ref[...] + 1.0
    return lax.rsqrt(deg), 1.0 / deg


def _tc_prescale_kernel(d0_ref, d1_ref, x_ref, g_ref):
    dinv, _ = _dinvs(d0_ref, d1_ref)
    g_ref[...] = x_ref[...] * dinv


def _tc_mid_kernel(a0_ref, a1_ref, x_ref, d0_ref, d1_ref,
                   w1_ref, b1_ref, w2_ref, t2_ref, g2_ref):
    dinv, dinv2 = _dinvs(d0_ref, d1_ref)
    p = dinv * (a0_ref[0] + a1_ref[0]) + dinv2 * x_ref[...]
    h = jnp.maximum(
        jnp.dot(p, w1_ref[...], preferred_element_type=jnp.float32)
        + b1_ref[...], 0.0)
    t2 = jnp.dot(h, w2_ref[...], preferred_element_type=jnp.float32)
    t2_ref[...] = t2
    g2_ref[...] = t2 * dinv


def _tc_final_kernel(a0_ref, a1_ref, t2_ref, d0_ref, d1_ref, b2_ref,
                     out_ref):
    dinv, dinv2 = _dinvs(d0_ref, d1_ref)
    o = dinv * (a0_ref[0] + a1_ref[0]) + dinv2 * t2_ref[...] + b2_ref[...]
    m = jnp.max(o, axis=1, keepdims=True)
    e = jnp.exp(o - m)
    lse = jnp.log(jnp.sum(e, axis=1, keepdims=True)) + m
    out_ref[...] = o - lse


def _row_spec(width):
    return pl.BlockSpec((ROWB, width), lambda i: (i, 0))


def _agg_spec(core):
    return pl.BlockSpec((1, ROWB, F), lambda i, core=core: (core, i, 0))


def _full_spec(shape):
    return pl.BlockSpec(shape, lambda i: tuple(0 for _ in shape))


_PAD = EPAD - E
_PADR = _PAD // C
_PAD3 = np.empty((_PADR, 2, C), dtype=np.int32)
_PAD3[:, 0, :] = (np.arange(_PAD, dtype=np.int32) % N).reshape(_PADR, C)
_PAD3[:, 1, :] = (N + np.arange(_PAD, dtype=np.int32)
                  % (NPAD - N)).reshape(_PADR, C)


def kernel(x, edge_index, W1, b1, W2, b2):
    ei3 = jnp.moveaxis(edge_index.astype(jnp.int32).reshape(2, E // C, C),
                       0, 1)
    e3 = jnp.concatenate([ei3, jnp.asarray(_PAD3)], axis=0)

    zeros1 = jnp.zeros((DSTRIPE,), jnp.float32)
    zeros2 = jnp.zeros((STRIPE, F), jnp.float32)
    ones_c = jnp.ones((128,), jnp.float32)

    degp = _sc_degree(e3, zeros1, ones_c)
    d0 = degp[0, :N].reshape(N, 1)
    d1 = degp[1, :N].reshape(N, 1)

    g1 = pl.pallas_call(
        _tc_prescale_kernel,
        grid=(GRID,),
        in_specs=[_row_spec(1), _row_spec(1), _row_spec(F)],
        out_specs=_row_spec(F),
        out_shape=jax.ShapeDtypeStruct((N, F), jnp.float32),
    )(d0, d1, x)

    agg1 = _sc_aggregate(g1, e3, zeros2)

    t2, g2 = pl.pallas_call(
        _tc_mid_kernel,
        grid=(GRID,),
        in_specs=[
            _agg_spec(0), _agg_spec(1), _row_spec(F),
            _row_spec(1), _row_spec(1),
            _full_spec((F, HID)), _full_spec((1, HID)), _full_spec((HID, F)),
        ],
        out_specs=[_row_spec(F), _row_spec(F)],
        out_shape=[
            jax.ShapeDtypeStruct((N, F), jnp.float32),
            jax.ShapeDtypeStruct((N, F), jnp.float32),
        ],
    )(agg1, agg1, x, d0, d1, W1, b1.reshape(1, HID), W2)

    agg2 = _sc_aggregate(g2, e3, zeros2)

    out = pl.pallas_call(
        _tc_final_kernel,
        grid=(GRID,),
        in_specs=[
            _agg_spec(0), _agg_spec(1), _row_spec(F),
            _row_spec(1), _row_spec(1), _full_spec((1, F)),
        ],
        out_specs=_row_spec(F),
        out_shape=jax.ShapeDtypeStruct((N, F), jnp.float32),
    )(agg2, agg2, t2, d0, d1, b2.reshape(1, F))

    return out
